# baseline (speedup 1.0000x reference)
import functools

import jax
import jax.numpy as jnp
from jax import lax
from jax.experimental import pallas as pl
from jax.experimental.pallas import tpu as pltpu
from jax.experimental.pallas import tpu_sc as plsc

N = 10000
E = 320000
D = 128

NC = 2
NS = 16
NW = NC * NS
EPT = E // NW
K = 40
NB = 252
EPTP = NB * K
RPT = 624
REM = N - NS * RPT


_BN = 1000


def _matmul_body(x_ref, w_ref, o_ref):
    o_ref[...] = jnp.dot(x_ref[...], w_ref[...],
                         preferred_element_type=jnp.float32)


def _matmul(x, W):
    return pl.pallas_call(
        _matmul_body,
        grid=(N // _BN,),
        in_specs=[
            pl.BlockSpec((_BN, D), lambda i: (i, 0)),
            pl.BlockSpec((D, D), lambda i: (0, 0)),
        ],
        out_specs=pl.BlockSpec((_BN, D), lambda i: (i, 0)),
        out_shape=jax.ShapeDtypeStruct((N, D), jnp.float32),
    )(x, W)


def _scale_rows(rows_v, w_all, bi):
    for g in range((K + 15) // 16):
        p = min(16, K - g * 16)
        lane0 = 16 - p
        wv = w_all[pl.ds(bi * K + g * 16 - lane0, 16)]
        for t in range(p):
            wj = wv[lane0 + t]
            j = g * 16 + t
            for cc in range(D // 16):
                sl = pl.ds(cc * 16, 16)
                rows_v[j, sl] = rows_v[j, sl] * wj


def _sc_body(support_hbm, src_hbm, dst_hbm, w_hbm, zeros_hbm, out_hbm,
             acc, src_all, dst_all, w_all, rows0, rows1, rows2,
             psem, g0, g1, g2, s0, s1, s2):
    c = lax.axis_index("c")
    s = lax.axis_index("s")
    wid = c * NS + s
    bufs = (rows0, rows1, rows2)
    gs = (g0, g1, g2)
    ss = (s0, s1, s2)

    d1 = pltpu.async_copy(src_hbm.at[wid], src_all, psem)
    d2 = pltpu.async_copy(dst_hbm.at[wid], dst_all, psem)
    d3 = pltpu.async_copy(w_hbm.at[wid], w_all, psem)
    d4 = pltpu.async_copy(zeros_hbm, acc.at[pl.ds(s * RPT, RPT)], psem)

    @pl.when(s == 0)
    def _():
        pltpu.sync_copy(zeros_hbm.at[pl.ds(0, REM)],
                        acc.at[pl.ds(NS * RPT, REM)])

    d1.wait()
    d2.wait()
    d3.wait()
    d4.wait()
    plsc.subcore_barrier()

    def gather(bi, buf, sem):
        return pltpu.async_copy(support_hbm.at[src_all.at[bi]], buf, sem)

    def drain(buf, sem):
        pltpu.make_async_copy(support_hbm.at[pl.ds(0, K)], buf, sem).wait()

    gather(0, rows0, g0)
    gather(1, rows1, g1)

    def triple(t, _):
        for u in range(3):
            j = 3 * t + u
            X, gx, sx = bufs[u], gs[u], ss[u]
            uz = (u + 2) % 3
            Z, gz, sz = bufs[uz], gs[uz], ss[uz]

            drain(X, gx)
            _scale_rows(X, w_all, j)

            @pl.when(j >= 1)
            def _():
                drain(Z, sz)

            @pl.when(j + 2 < NB)
            def _():
                gather(j + 2, Z, gz)

            pltpu.async_copy(X, acc.at[dst_all.at[j]], sx, add=True)
        return 0

    lax.fori_loop(0, NB // 3, triple, 0)
    drain(bufs[(NB - 1) % 3], ss[(NB - 1) % 3])
    plsc.subcore_barrier()

    pltpu.sync_copy(acc.at[pl.ds(s * RPT, RPT)],
                    out_hbm.at[c, pl.ds(s * RPT, RPT)])

    @pl.when(s == 0)
    def _():
        pltpu.sync_copy(acc.at[pl.ds(NS * RPT, REM)],
                        out_hbm.at[c, pl.ds(NS * RPT, REM)])


def _sc_aggregate(support, src, dst, w):
    mesh = plsc.VectorSubcoreMesh(core_axis_name="c", subcore_axis_name="s",
                                  num_cores=NC, num_subcores=NS)
    return pl.kernel(
        _sc_body,
        out_type=jax.ShapeDtypeStruct((NC, N, D), jnp.float32),
        mesh=mesh,
        compiler_params=pltpu.CompilerParams(use_tc_tiling_on_sc=False),
        scratch_types=[
            pltpu.VMEM_SHARED((N, D), jnp.float32),
            pltpu.VMEM((NB, K), jnp.int32),
            pltpu.VMEM((NB, K), jnp.int32),
            pltpu.VMEM((EPTP,), jnp.float32),
            pltpu.VMEM((K, D), jnp.float32),
            pltpu.VMEM((K, D), jnp.float32),
            pltpu.VMEM((K, D), jnp.float32),
            pltpu.SemaphoreType.DMA,
            pltpu.SemaphoreType.DMA,
            pltpu.SemaphoreType.DMA,
            pltpu.SemaphoreType.DMA,
            pltpu.SemaphoreType.DMA,
            pltpu.SemaphoreType.DMA,
            pltpu.SemaphoreType.DMA,
        ],
    )(support.reshape(N, D),
      _pad_meta(src).reshape(NW, NB, K),
      _pad_meta(dst).reshape(NW, NB, K),
      _pad_meta(w).reshape(NW, EPTP),
      jnp.zeros((RPT, D), jnp.float32))


def _pad_meta(a):
    return jnp.pad(a.reshape(NW, EPT), ((0, 0), (0, EPTP - EPT)))


def _combine_body(p_ref, b_ref, o_ref):
    o_ref[...] = p_ref[0] + p_ref[1] + b_ref[...]


def _combine(partials, b):
    return pl.pallas_call(
        _combine_body,
        grid=(N // _BN,),
        in_specs=[
            pl.BlockSpec((NC, _BN, D), lambda i: (0, i, 0)),
            pl.BlockSpec((1, D), lambda i: (0, 0)),
        ],
        out_specs=pl.BlockSpec((_BN, D), lambda i: (i, 0)),
        out_shape=jax.ShapeDtypeStruct((N, D), jnp.float32),
    )(partials, b.reshape(1, D))


def kernel(input, edge_index, edge_weight, W, b):
    support = _matmul(input, W)
    partials = _sc_aggregate(support, edge_index[0], edge_index[1],
                             edge_weight)
    return _combine(partials, b)


# speedup vs baseline: 1.4638x; 1.4638x over previous
import functools

import jax
import jax.numpy as jnp
from jax import lax
from jax.experimental import pallas as pl
from jax.experimental.pallas import tpu as pltpu
from jax.experimental.pallas import tpu_sc as plsc

N = 10000
E = 320000
D = 128

NC = 2
NS = 16
NW = NC * NS
EPT = E // NW
K = 40
NB = 250
EPTP = NB * K
RPT = 624
REM = N - NS * RPT


_BN = 1000


def _matmul_body(x_ref, w_ref, o_ref):
    o_ref[...] = jnp.dot(x_ref[...], w_ref[...],
                         preferred_element_type=jnp.float32)


def _matmul(x, W):
    return pl.pallas_call(
        _matmul_body,
        grid=(N // _BN,),
        in_specs=[
            pl.BlockSpec((_BN, D), lambda i: (i, 0)),
            pl.BlockSpec((D, D), lambda i: (0, 0)),
        ],
        out_specs=pl.BlockSpec((_BN, D), lambda i: (i, 0)),
        out_shape=jax.ShapeDtypeStruct((N, D), jnp.float32),
    )(x, W)


def _scale_rows(rows_v, w_all, bi):
    for g in range((K + 15) // 16):
        p = min(16, K - g * 16)
        lane0 = 16 - p
        wv = w_all[pl.ds(bi * K + g * 16 - lane0, 16)]
        for t in range(p):
            wj = wv[lane0 + t]
            j = g * 16 + t
            for cc in range(D // 16):
                sl = pl.ds(cc * 16, 16)
                rows_v[j, sl] = rows_v[j, sl] * wj


def _sc_body(support_hbm, src_hbm, dst_hbm, w_hbm, zeros_hbm, out_hbm,
             acc, src_all, dst_all, w_all, rows0, rows1, rows2,
             psem, g0, g1, g2, s0, s1, s2):
    c = lax.axis_index("c")
    s = lax.axis_index("s")
    wid = c * NS + s
    bufs = (rows0, rows1, rows2)
    gs = (g0, g1, g2)
    ss = (s0, s1, s2)

    d1 = pltpu.async_copy(src_hbm.at[wid], src_all, psem)
    d2 = pltpu.async_copy(dst_hbm.at[wid], dst_all, psem)
    d3 = pltpu.async_copy(w_hbm.at[wid], w_all, psem)
    d4 = pltpu.async_copy(zeros_hbm, acc.at[pl.ds(s * RPT, RPT)], psem)

    @pl.when(s == 0)
    def _():
        pltpu.sync_copy(zeros_hbm.at[pl.ds(0, REM)],
                        acc.at[pl.ds(NS * RPT, REM)])

    d1.wait()
    d2.wait()
    d3.wait()
    d4.wait()
    plsc.subcore_barrier()

    def gather(bi, buf, sem):
        return pltpu.async_copy(support_hbm.at[src_all.at[bi]], buf, sem)

    def drain(buf, sem):
        pltpu.make_async_copy(support_hbm.at[pl.ds(0, K)], buf, sem).wait()

    gather(0, rows0, g0)
    gather(1, rows1, g1)

    drain(rows0, g0)
    _scale_rows(rows0, w_all, 0)
    gather(2, rows2, g2)
    pltpu.async_copy(rows0, acc.at[dst_all.at[0]], s0, add=True)

    def triple(t, _):
        for u in range(3):
            j = 3 * t + u + 1
            X, gx, sx = bufs[(u + 1) % 3], gs[(u + 1) % 3], ss[(u + 1) % 3]
            uz = u
            Z, gz, sz = bufs[uz], gs[uz], ss[uz]

            drain(X, gx)
            _scale_rows(X, w_all, j)
            drain(Z, sz)

            @pl.when(j + 2 < NB)
            def _():
                gather(j + 2, Z, gz)

            pltpu.async_copy(X, acc.at[dst_all.at[j]], sx, add=True)
        return 0

    lax.fori_loop(0, (NB - 1) // 3, triple, 0)
    drain(bufs[(NB - 1) % 3], ss[(NB - 1) % 3])
    plsc.subcore_barrier()

    pltpu.sync_copy(acc.at[pl.ds(s * RPT, RPT)],
                    out_hbm.at[c, pl.ds(s * RPT, RPT)])

    @pl.when(s == 0)
    def _():
        pltpu.sync_copy(acc.at[pl.ds(NS * RPT, REM)],
                        out_hbm.at[c, pl.ds(NS * RPT, REM)])


def _sc_aggregate(support, src, dst, w):
    mesh = plsc.VectorSubcoreMesh(core_axis_name="c", subcore_axis_name="s",
                                  num_cores=NC, num_subcores=NS)
    return pl.kernel(
        _sc_body,
        out_type=jax.ShapeDtypeStruct((NC, N, D), jnp.float32),
        mesh=mesh,
        compiler_params=pltpu.CompilerParams(use_tc_tiling_on_sc=False),
        scratch_types=[
            pltpu.VMEM_SHARED((N, D), jnp.float32),
            pltpu.VMEM((NB, K), jnp.int32),
            pltpu.VMEM((NB, K), jnp.int32),
            pltpu.VMEM((EPTP,), jnp.float32),
            pltpu.VMEM((K, D), jnp.float32),
            pltpu.VMEM((K, D), jnp.float32),
            pltpu.VMEM((K, D), jnp.float32),
            pltpu.SemaphoreType.DMA,
            pltpu.SemaphoreType.DMA,
            pltpu.SemaphoreType.DMA,
            pltpu.SemaphoreType.DMA,
            pltpu.SemaphoreType.DMA,
            pltpu.SemaphoreType.DMA,
            pltpu.SemaphoreType.DMA,
        ],
    )(support.reshape(N, D),
      _pad_meta(src).reshape(NW, NB, K),
      _pad_meta(dst).reshape(NW, NB, K),
      _pad_meta(w).reshape(NW, EPTP),
      jnp.zeros((RPT, D), jnp.float32))


def _pad_meta(a):
    return jnp.pad(a.reshape(NW, EPT), ((0, 0), (0, EPTP - EPT)))


def _combine_body(p_ref, b_ref, o_ref):
    o_ref[...] = p_ref[0] + p_ref[1] + b_ref[...]


def _combine(partials, b):
    return pl.pallas_call(
        _combine_body,
        grid=(N // _BN,),
        in_specs=[
            pl.BlockSpec((NC, _BN, D), lambda i: (0, i, 0)),
            pl.BlockSpec((1, D), lambda i: (0, 0)),
        ],
        out_specs=pl.BlockSpec((_BN, D), lambda i: (i, 0)),
        out_shape=jax.ShapeDtypeStruct((N, D), jnp.float32),
    )(partials, b.reshape(1, D))


def kernel(input, edge_index, edge_weight, W, b):
    support = _matmul(input, W)
    partials = _sc_aggregate(support, edge_index[0], edge_index[1],
                             edge_weight)
    return _combine(partials, b)
